# Initial kernel scaffold; baseline (speedup 1.0000x reference)
import functools

import jax
import jax.numpy as jnp
from jax import lax
from jax.experimental import pallas as pl
from jax.experimental.pallas import tpu as pltpu
from jax.experimental.pallas import tpu_sc as plsc

N = 10000
E = 320000
D = 128
H = 128
C = 64

NC = 2
NS = 16
NW = NC * NS
EPT = E // NW
BATCH = 80
NB = EPT // BATCH
ROWS_PT = N // NS


def _make_seg_sum(w: int):
  mesh = plsc.VectorSubcoreMesh(core_axis_name="c", subcore_axis_name="s")

  @functools.partial(
      pl.kernel,
      out_type=jax.ShapeDtypeStruct((NC, N, w), jnp.float32),
      mesh=mesh,
      scratch_types=[
          pltpu.VMEM((NB, BATCH), jnp.int32),
          pltpu.VMEM((NB, BATCH), jnp.int32),
          pltpu.VMEM((2, BATCH, w), jnp.float32),
          pltpu.SemaphoreType.DMA((2,)),
          pltpu.VMEM_SHARED((N, w), jnp.float32),
      ],
  )
  def seg(src_hbm, dst_hbm, p_hbm, zeros_hbm, out_hbm,
          src_v, dst_v, buf, sems, acc):
    c = lax.axis_index("c")
    s = lax.axis_index("s")
    wid = s * NC + c
    r0 = s * ROWS_PT

    pltpu.sync_copy(zeros_hbm.at[pl.ds(r0, ROWS_PT)], acc.at[pl.ds(r0, ROWS_PT)])
    pltpu.sync_copy(src_hbm.at[wid], src_v)
    pltpu.sync_copy(dst_hbm.at[wid], dst_v)
    plsc.subcore_barrier()

    pltpu.async_copy(p_hbm.at[src_v.at[0]], buf.at[0], sems.at[0])

    def body(j, carry):
      nxt = j + 1

      @pl.when(nxt < NB)
      def _():
        pltpu.async_copy(p_hbm.at[src_v.at[nxt]], buf.at[nxt % 2],
                         sems.at[nxt % 2])

      pltpu.make_async_copy(p_hbm.at[src_v.at[j]], buf.at[j % 2],
                            sems.at[j % 2]).wait()
      pltpu.sync_copy(buf.at[j % 2], acc.at[dst_v.at[j]], add=True)
      return carry

    lax.fori_loop(0, NB, body, 0)

    plsc.subcore_barrier()
    pltpu.sync_copy(acc.at[pl.ds(r0, ROWS_PT)],
                    out_hbm.at[c, pl.ds(r0, ROWS_PT)])

  return seg


_seg136 = _make_seg_sum(H + 8)
_seg128 = _make_seg_sum(H)
_seg64 = _make_seg_sum(C)

_R = 1000
_G = N // _R


def _mm0_body(x_ref, w_ref, b_ref, s_ref, p_ref):
  y = jnp.dot(x_ref[...], w_ref[...], preferred_element_type=jnp.float32)
  s_ref[...] = y[:, :H] + b_ref[...]
  p_ref[:, :H] = y[:, H:]
  p_ref[:, H:] = jnp.ones((_R, 8), jnp.float32)


def _stage1_body(s0_ref, g_ref, w_ref, b_ref, s1_ref, p1_ref, inv_ref):
  g0 = g_ref[0]
  g1 = g_ref[1]
  deg = jnp.maximum(g0[:, H:H + 1] + g1[:, H:H + 1], 1.0)
  inv = 1.0 / deg
  h = jnp.maximum(s0_ref[...] + (g0[:, :H] + g1[:, :H]) * inv, 0.0)
  y = jnp.dot(h, w_ref[...], preferred_element_type=jnp.float32)
  s1_ref[...] = y[:, :H] + b_ref[...]
  p1_ref[...] = y[:, H:]
  inv_ref[...] = jnp.broadcast_to(inv, (_R, 8))


def _stage2_body(s1_ref, g_ref, inv_ref, w_ref, b_ref, s2_ref, p2_ref):
  inv = inv_ref[:, 0:1]
  h = jnp.maximum(s1_ref[...] + (g_ref[0] + g_ref[1]) * inv, 0.0)
  y = jnp.dot(h, w_ref[...], preferred_element_type=jnp.float32)
  s2_ref[...] = y[:, :C] + b_ref[...]
  p2_ref[...] = y[:, C:]


def _stage3_body(s2_ref, g_ref, inv_ref, out_ref):
  inv = inv_ref[:, 0:1]
  out_ref[...] = jnp.maximum(s2_ref[...] + (g_ref[0] + g_ref[1]) * inv, 0.0)


def _rows(w):
  return pl.BlockSpec((_R, w), lambda i: (i, 0))


def _pair(w):
  return pl.BlockSpec((NC, _R, w), lambda i: (0, i, 0))


def _whole(a, b):
  return pl.BlockSpec((a, b), lambda i: (0, 0))


def kernel(x, edge_index, W_self0, W_neigh0, b0, W_self1, W_neigh1, b1,
           W_self2, W_neigh2, b2):
  src = edge_index[0].reshape(NW, NB, BATCH)
  dst = edge_index[1].reshape(NW, NB, BATCH)
  z136 = jnp.zeros((N, H + 8), jnp.float32)
  z128 = jnp.zeros((N, H), jnp.float32)
  z64 = jnp.zeros((N, C), jnp.float32)
  wc0 = jnp.concatenate([W_self0, W_neigh0], axis=1)
  wc1 = jnp.concatenate([W_self1, W_neigh1], axis=1)
  wc2 = jnp.concatenate([W_self2, W_neigh2], axis=1)

  s0, p0 = pl.pallas_call(
      _mm0_body,
      grid=(_G,),
      in_specs=[_rows(D), _whole(D, 2 * H), _whole(1, H)],
      out_specs=[_rows(H), _rows(H + 8)],
      out_shape=[jax.ShapeDtypeStruct((N, H), jnp.float32),
                 jax.ShapeDtypeStruct((N, H + 8), jnp.float32)],
  )(x, wc0, b0.reshape(1, H))

  g0 = _seg136(src, dst, p0, z136)

  s1, p1, invd = pl.pallas_call(
      _stage1_body,
      grid=(_G,),
      in_specs=[_rows(H), _pair(H + 8), _whole(H, 2 * H), _whole(1, H)],
      out_specs=[_rows(H), _rows(H), _rows(8)],
      out_shape=[jax.ShapeDtypeStruct((N, H), jnp.float32),
                 jax.ShapeDtypeStruct((N, H), jnp.float32),
                 jax.ShapeDtypeStruct((N, 8), jnp.float32)],
  )(s0, g0, wc1, b1.reshape(1, H))

  g1 = _seg128(src, dst, p1, z128)

  s2, p2 = pl.pallas_call(
      _stage2_body,
      grid=(_G,),
      in_specs=[_rows(H), _pair(H), _rows(8), _whole(H, 2 * C), _whole(1, C)],
      out_specs=[_rows(C), _rows(C)],
      out_shape=[jax.ShapeDtypeStruct((N, C), jnp.float32),
                 jax.ShapeDtypeStruct((N, C), jnp.float32)],
  )(s1, g1, invd, wc2, b2.reshape(1, C))

  g2 = _seg64(src, dst, p2, z64)

  out = pl.pallas_call(
      _stage3_body,
      grid=(_G,),
      in_specs=[_rows(C), _pair(C), _rows(8)],
      out_specs=_rows(C),
      out_shape=jax.ShapeDtypeStruct((N, C), jnp.float32),
  )(s2, g2, invd)

  return out


# speedup vs baseline: 10.2668x; 10.2668x over previous
import functools

import jax
import jax.numpy as jnp
from jax import lax
from jax.experimental import pallas as pl
from jax.experimental.pallas import tpu as pltpu
from jax.experimental.pallas import tpu_sc as plsc

N = 10000
E = 320000
D = 128
H = 128
C = 64

NC = 2
NS = 16
NW = NC * NS
EPT = E // NW
BATCH = 80
NB = EPT // BATCH
ROWS_PT = 624
TAIL0 = ROWS_PT * NS
TAIL = N - TAIL0


def _make_seg_sum(w: int):
  mesh = plsc.VectorSubcoreMesh(core_axis_name="c", subcore_axis_name="s")

  @functools.partial(
      pl.kernel,
      out_type=jax.ShapeDtypeStruct((NC, N, w), jnp.float32),
      mesh=mesh,
      compiler_params=pltpu.CompilerParams(use_tc_tiling_on_sc=False),
      scratch_types=[
          pltpu.VMEM((NB, BATCH), jnp.int32),
          pltpu.VMEM((NB, BATCH), jnp.int32),
          pltpu.VMEM((2, BATCH, w), jnp.float32),
          pltpu.SemaphoreType.DMA((2,)),
          pltpu.VMEM_SHARED((N, w), jnp.float32),
      ],
  )
  def seg(src_hbm, dst_hbm, p_hbm, zeros_hbm, out_hbm,
          src_v, dst_v, buf, sems, acc):
    c = lax.axis_index("c")
    s = lax.axis_index("s")
    wid = s * NC + c
    r0 = s * ROWS_PT

    pltpu.sync_copy(zeros_hbm.at[pl.ds(r0, ROWS_PT)], acc.at[pl.ds(r0, ROWS_PT)])

    @pl.when(s == NS - 1)
    def _():
      pltpu.sync_copy(zeros_hbm.at[pl.ds(TAIL0, TAIL)],
                      acc.at[pl.ds(TAIL0, TAIL)])

    pltpu.sync_copy(src_hbm.at[wid], src_v)
    pltpu.sync_copy(dst_hbm.at[wid], dst_v)
    plsc.subcore_barrier()

    pltpu.async_copy(p_hbm.at[src_v.at[0]], buf.at[0], sems.at[0])

    def body(j, carry):
      nxt = j + 1

      @pl.when(nxt < NB)
      def _():
        pltpu.async_copy(p_hbm.at[src_v.at[nxt]], buf.at[nxt % 2],
                         sems.at[nxt % 2])

      pltpu.make_async_copy(p_hbm.at[src_v.at[j]], buf.at[j % 2],
                            sems.at[j % 2]).wait()
      pltpu.sync_copy(buf.at[j % 2], acc.at[dst_v.at[j]], add=True)
      return carry

    lax.fori_loop(0, NB, body, 0)

    plsc.subcore_barrier()
    pltpu.sync_copy(acc.at[pl.ds(r0, ROWS_PT)],
                    out_hbm.at[c, pl.ds(r0, ROWS_PT)])

    @pl.when(s == NS - 1)
    def _():
      pltpu.sync_copy(acc.at[pl.ds(TAIL0, TAIL)],
                      out_hbm.at[c, pl.ds(TAIL0, TAIL)])

  return seg


_seg136 = _make_seg_sum(H + 8)
_seg128 = _make_seg_sum(H)
_seg64 = _make_seg_sum(C)

_R = 1000
_G = N // _R


def _mm0_body(x_ref, w_ref, b_ref, s_ref, p_ref):
  y = jnp.dot(x_ref[...], w_ref[...], preferred_element_type=jnp.float32)
  s_ref[...] = y[:, :H] + b_ref[...]
  p_ref[:, :H] = y[:, H:]
  p_ref[:, H:] = jnp.ones((_R, 8), jnp.float32)


def _stage1_body(s0_ref, g_ref, w_ref, b_ref, s1_ref, p1_ref, inv_ref):
  g0 = g_ref[0]
  g1 = g_ref[1]
  deg = jnp.maximum(g0[:, H:H + 1] + g1[:, H:H + 1], 1.0)
  inv = 1.0 / deg
  h = jnp.maximum(s0_ref[...] + (g0[:, :H] + g1[:, :H]) * inv, 0.0)
  y = jnp.dot(h, w_ref[...], preferred_element_type=jnp.float32)
  s1_ref[...] = y[:, :H] + b_ref[...]
  p1_ref[...] = y[:, H:]
  inv_ref[...] = jnp.broadcast_to(inv, (_R, 8))


def _stage2_body(s1_ref, g_ref, inv_ref, w_ref, b_ref, s2_ref, p2_ref):
  inv = inv_ref[:, 0:1]
  h = jnp.maximum(s1_ref[...] + (g_ref[0] + g_ref[1]) * inv, 0.0)
  y = jnp.dot(h, w_ref[...], preferred_element_type=jnp.float32)
  s2_ref[...] = y[:, :C] + b_ref[...]
  p2_ref[...] = y[:, C:]


def _stage3_body(s2_ref, g_ref, inv_ref, out_ref):
  inv = inv_ref[:, 0:1]
  out_ref[...] = jnp.maximum(s2_ref[...] + (g_ref[0] + g_ref[1]) * inv, 0.0)


def _rows(w):
  return pl.BlockSpec((_R, w), lambda i: (i, 0))


def _pair(w):
  return pl.BlockSpec((NC, _R, w), lambda i: (0, i, 0))


def _whole(a, b):
  return pl.BlockSpec((a, b), lambda i: (0, 0))


def kernel(x, edge_index, W_self0, W_neigh0, b0, W_self1, W_neigh1, b1,
           W_self2, W_neigh2, b2):
  src = edge_index[0].reshape(NW, NB, BATCH)
  dst = edge_index[1].reshape(NW, NB, BATCH)
  z136 = jnp.zeros((N, H + 8), jnp.float32)
  z128 = jnp.zeros((N, H), jnp.float32)
  z64 = jnp.zeros((N, C), jnp.float32)
  wc0 = jnp.concatenate([W_self0, W_neigh0], axis=1)
  wc1 = jnp.concatenate([W_self1, W_neigh1], axis=1)
  wc2 = jnp.concatenate([W_self2, W_neigh2], axis=1)

  s0, p0 = pl.pallas_call(
      _mm0_body,
      grid=(_G,),
      in_specs=[_rows(D), _whole(D, 2 * H), _whole(1, H)],
      out_specs=[_rows(H), _rows(H + 8)],
      out_shape=[jax.ShapeDtypeStruct((N, H), jnp.float32),
                 jax.ShapeDtypeStruct((N, H + 8), jnp.float32)],
  )(x, wc0, b0.reshape(1, H))

  g0 = _seg136(src, dst, p0, z136)

  s1, p1, invd = pl.pallas_call(
      _stage1_body,
      grid=(_G,),
      in_specs=[_rows(H), _pair(H + 8), _whole(H, 2 * H), _whole(1, H)],
      out_specs=[_rows(H), _rows(H), _rows(8)],
      out_shape=[jax.ShapeDtypeStruct((N, H), jnp.float32),
                 jax.ShapeDtypeStruct((N, H), jnp.float32),
                 jax.ShapeDtypeStruct((N, 8), jnp.float32)],
  )(s0, g0, wc1, b1.reshape(1, H))

  g1 = _seg128(src, dst, p1, z128)

  s2, p2 = pl.pallas_call(
      _stage2_body,
      grid=(_G,),
      in_specs=[_rows(H), _pair(H), _rows(8), _whole(H, 2 * C), _whole(1, C)],
      out_specs=[_rows(C), _rows(C)],
      out_shape=[jax.ShapeDtypeStruct((N, C), jnp.float32),
                 jax.ShapeDtypeStruct((N, C), jnp.float32)],
  )(s1, g1, invd, wc2, b2.reshape(1, C))

  g2 = _seg64(src, dst, p2, z64)

  out = pl.pallas_call(
      _stage3_body,
      grid=(_G,),
      in_specs=[_rows(C), _pair(C), _rows(8)],
      out_specs=_rows(C),
      out_shape=jax.ShapeDtypeStruct((N, C), jnp.float32),
  )(s2, g2, invd)

  return out
